# Initial kernel scaffold; baseline (speedup 1.0000x reference)
import jax
import jax.numpy as jnp
from jax.experimental import pallas as pl


def kernel(y_1, t):
    raise NotImplementedError("write your pallas kernel here")


# speedup vs baseline: 16.5162x; 16.5162x over previous
import functools

import jax
import jax.numpy as jnp
from jax.experimental import pallas as pl


def _rows_body(K, t_ref, y_ref, h_ref, s_ref, w_ref):
    y = y_ref[...]
    tcol = t_ref[...]
    BR = y.shape[0]
    cols = jax.lax.broadcasted_iota(jnp.int32, (BR, K), 1)
    L1 = jnp.sum(jnp.where(cols == tcol, y, 0.0), axis=1, keepdims=True)
    M0 = jnp.max(y, axis=1, keepdims=True)
    idx0 = jnp.min(jnp.where(y == M0, cols, K), axis=1, keepdims=True)
    M1 = jnp.max(jnp.where(cols == idx0, -jnp.inf, y), axis=1, keepdims=True)
    lse = jnp.log(jnp.sum(jnp.exp(y - M0), axis=1, keepdims=True)) + M0
    f1 = idx0 == tcol
    h_ref[0] = jnp.maximum(1.0 - L1 + jnp.where(f1, M1, M0), 0.0)
    s_ref[0] = jnp.maximum(1.0 - L1 + jnp.where(f1, M1, lse), 0.0)
    w_ref[0] = jnp.where(f1, 0.0, 1.0)


def _sel_body(N, BR, NB, h_ref, s_ref, w_ref, out_ref):
    h = h_ref[...]
    s = s_ref[...]
    E = jnp.sum(w_ref[...])
    C = jnp.float32(N) + E
    hb = jax.lax.bitcast_convert_type(h, jnp.int32)
    ridx = jax.lax.broadcasted_iota(jnp.int32, (NB, BR, 1), 1)
    bidx = jax.lax.broadcasted_iota(jnp.int32, (NB, BR, 1), 0)
    idx = bidx * BR + ridx

    def cnt_lt(v):
        return jnp.sum(jnp.where(hb < v, 1.0, 0.0))

    def sum_h_lt(v):
        return jnp.sum(jnp.where(hb < v, h, 0.0))

    def ph1(b, v):
        cand = v + jnp.left_shift(jnp.int32(1), 30 - b)
        ok = sum_h_lt(cand) + cnt_lt(cand) - 1.0 <= C
        return jnp.where(ok, cand, v)

    vstar = jax.lax.fori_loop(0, 31, ph1, jnp.int32(0))
    hval = jax.lax.bitcast_convert_type(vstar, jnp.float32)
    n_lt = cnt_lt(vstar)
    s_lt = sum_h_lt(vstar)
    cnt_tie = jnp.sum(jnp.where(hb == vstar, 1.0, 0.0))
    m = jnp.floor((C + 1.0 - n_lt - s_lt) / (hval + 1.0))
    m = jnp.clip(m, 0.0, cnt_tie)
    kstar = n_lt + m
    Sstar = s_lt + m * hval
    total = jnp.sum(h)
    upb = jnp.where(kstar == 0.0, total <= C, Sstar <= C - kstar)
    kf = jnp.minimum(kstar + jnp.where(upb, 1.0, 0.0), jnp.float32(N))

    def ph2(b, v):
        cand = v + jnp.left_shift(jnp.int32(1), 30 - b)
        ok = cnt_lt(cand) < kf
        return jnp.where(ok, cand, v)

    w = jax.lax.fori_loop(0, 31, ph2, jnp.int32(0))
    m2 = kf - cnt_lt(w)
    sum_s_lt = jnp.sum(jnp.where(hb < w, s, 0.0))
    tie = hb == w

    def ph3(_, lo_hi):
        lo, hi = lo_hi
        mid = lo + (hi - lo + 1) // 2
        ok = jnp.sum(jnp.where(tie & (idx <= mid), 1.0, 0.0)) < m2
        return jnp.where(ok, mid, lo), jnp.where(ok, hi, mid - 1)

    jmax, _ = jax.lax.fori_loop(0, 14, ph3, (jnp.int32(-1), jnp.int32(N - 1)))
    sum_s_tie = jnp.sum(jnp.where(tie & (idx <= jmax + 1), s, 0.0))
    res = (sum_s_lt + sum_s_tie) / kf
    out_ref[...] = jnp.full((1, 1), res, dtype=jnp.float32)


def _impl(y_1, t, interpret=False):
    N, K = y_1.shape
    BR = 512
    NB = N // BR
    t2 = t.reshape(N, 1)
    h_all, s_all, w_all = pl.pallas_call(
        functools.partial(_rows_body, K),
        grid=(NB,),
        in_specs=[
            pl.BlockSpec((BR, 1), lambda i: (i, 0)),
            pl.BlockSpec((BR, K), lambda i: (i, 0)),
        ],
        out_specs=[pl.BlockSpec((1, BR, 1), lambda i: (i, 0, 0))] * 3,
        out_shape=[jax.ShapeDtypeStruct((NB, BR, 1), jnp.float32)] * 3,
        interpret=interpret,
    )(t2, y_1)
    out = pl.pallas_call(
        functools.partial(_sel_body, N, BR, NB),
        out_shape=jax.ShapeDtypeStruct((1, 1), jnp.float32),
        interpret=interpret,
    )(h_all, s_all, w_all)
    return out[0, 0]


def kernel(y_1, t):
    return _impl(y_1, t)


# speedup vs baseline: 34.6153x; 2.0958x over previous
import functools

import jax
import jax.numpy as jnp
from jax.experimental import pallas as pl


def _rows_body(K, t_ref, y_ref, h_ref, s_ref, w_ref):
    y = y_ref[...]
    tcol = t_ref[...]
    BR = y.shape[0]
    cols = jax.lax.broadcasted_iota(jnp.int32, (BR, K), 1)
    L1 = jnp.sum(jnp.where(cols == tcol, y, 0.0), axis=1, keepdims=True)
    M0 = jnp.max(y, axis=1, keepdims=True)
    idx0 = jnp.min(jnp.where(y == M0, cols, K), axis=1, keepdims=True)
    M1 = jnp.max(jnp.where(cols == idx0, -jnp.inf, y), axis=1, keepdims=True)
    lse = jnp.log(jnp.sum(jnp.exp(y - M0), axis=1, keepdims=True)) + M0
    f1 = idx0 == tcol
    h_ref[0] = jnp.maximum(1.0 - L1 + jnp.where(f1, M1, M0), 0.0)
    s_ref[0] = jnp.maximum(1.0 - L1 + jnp.where(f1, M1, lse), 0.0)
    w_ref[0] = jnp.where(f1, 0.0, 1.0)


def _sel_body(N, R, L, h_ref, s_ref, w_ref, out_ref):
    h = h_ref[...]
    s = s_ref[...]
    E = jnp.sum(w_ref[...])
    C = jnp.float32(N) + E
    hb = jax.lax.bitcast_convert_type(h, jnp.int32)
    aidx = jax.lax.broadcasted_iota(jnp.int32, (R, L), 0)
    bidx = jax.lax.broadcasted_iota(jnp.int32, (R, L), 1)
    idx = aidx * L + bidx

    def cnt_lt(v):
        return jnp.sum(jnp.where(hb < v, 1.0, 0.0))

    def sum_h_lt(v):
        return jnp.sum(jnp.where(hb < v, h, 0.0))

    def ph1(b, v):
        cand = v + jnp.left_shift(jnp.int32(1), 30 - b)
        ok = sum_h_lt(cand) + cnt_lt(cand) - 1.0 <= C
        return jnp.where(ok, cand, v)

    vstar = jax.lax.fori_loop(0, 31, ph1, jnp.int32(0))
    hval = jax.lax.bitcast_convert_type(vstar, jnp.float32)
    n_lt = cnt_lt(vstar)
    s_lt = sum_h_lt(vstar)
    cnt_tie = jnp.sum(jnp.where(hb == vstar, 1.0, 0.0))
    m = jnp.floor((C + 1.0 - n_lt - s_lt) / (hval + 1.0))
    m = jnp.clip(m, 0.0, cnt_tie)
    kstar = n_lt + m
    Sstar = s_lt + m * hval
    total = jnp.sum(h)
    upb = jnp.where(kstar == 0.0, total <= C, Sstar <= C - kstar)
    kf = jnp.minimum(kstar + jnp.where(upb, 1.0, 0.0), jnp.float32(N))

    def ph2(b, v):
        cand = v + jnp.left_shift(jnp.int32(1), 30 - b)
        ok = cnt_lt(cand) < kf
        return jnp.where(ok, cand, v)

    w = jax.lax.fori_loop(0, 31, ph2, jnp.int32(0))
    m2 = kf - cnt_lt(w)
    sum_s_lt = jnp.sum(jnp.where(hb < w, s, 0.0))
    tie = hb == w

    def ph3(_, lo_hi):
        lo, hi = lo_hi
        mid = lo + (hi - lo + 1) // 2
        ok = jnp.sum(jnp.where(tie & (idx <= mid), 1.0, 0.0)) < m2
        return jnp.where(ok, mid, lo), jnp.where(ok, hi, mid - 1)

    jmax, _ = jax.lax.fori_loop(0, 14, ph3, (jnp.int32(-1), jnp.int32(N - 1)))
    sum_s_tie = jnp.sum(jnp.where(tie & (idx <= jmax + 1), s, 0.0))
    res = (sum_s_lt + sum_s_tie) / kf
    out_ref[...] = jnp.full((1, 1), res, dtype=jnp.float32)


def _impl(y_1, t, interpret=False):
    N, K = y_1.shape
    BR = 512
    NB = N // BR
    t2 = t.reshape(N, 1)
    h_all, s_all, w_all = pl.pallas_call(
        functools.partial(_rows_body, K),
        grid=(NB,),
        in_specs=[
            pl.BlockSpec((BR, 1), lambda i: (i, 0)),
            pl.BlockSpec((BR, K), lambda i: (i, 0)),
        ],
        out_specs=[pl.BlockSpec((1, BR, 1), lambda i: (i, 0, 0))] * 3,
        out_shape=[jax.ShapeDtypeStruct((NB, BR, 1), jnp.float32)] * 3,
        interpret=interpret,
    )(t2, y_1)
    R, L = N // 128, 128
    h2 = h_all.reshape(R, L)
    s2 = s_all.reshape(R, L)
    w2 = w_all.reshape(R, L)
    out = pl.pallas_call(
        functools.partial(_sel_body, N, R, L),
        out_shape=jax.ShapeDtypeStruct((1, 1), jnp.float32),
        interpret=interpret,
    )(h2, s2, w2)
    return out[0, 0]


def kernel(y_1, t):
    return _impl(y_1, t)


# speedup vs baseline: 35.8290x; 1.0351x over previous
import functools

import jax
import jax.numpy as jnp
from jax.experimental import pallas as pl


def _rows_body(K, t_ref, y_ref, h_ref, s_ref, w_ref):
    y = y_ref[...]
    tcol = t_ref[...]
    BR = y.shape[0]
    cols = jax.lax.broadcasted_iota(jnp.int32, (BR, K), 1)
    eqm = cols == tcol
    L1 = jnp.sum(jnp.where(eqm, y, 0.0), axis=1, keepdims=True)
    M0 = jnp.max(y, axis=1, keepdims=True)
    M1 = jnp.max(jnp.where(eqm, -jnp.inf, y), axis=1, keepdims=True)
    lse = jnp.log(jnp.sum(jnp.exp(y - M0), axis=1, keepdims=True)) + M0
    f1 = L1 == M0
    h_ref[0] = jnp.maximum(1.0 - L1 + jnp.where(f1, M1, M0), 0.0)
    s_ref[0] = jnp.maximum(1.0 - L1 + jnp.where(f1, M1, lse), 0.0)
    w_ref[0] = jnp.where(f1, 0.0, 1.0)


def _sel_body(N, R, L, h_ref, s_ref, w_ref, out_ref):
    h = h_ref[...]
    s = s_ref[...]
    E = jnp.sum(w_ref[...])
    C = jnp.float32(N) + E
    hb = jax.lax.bitcast_convert_type(h, jnp.int32)
    aidx = jax.lax.broadcasted_iota(jnp.int32, (R, L), 0)
    bidx = jax.lax.broadcasted_iota(jnp.int32, (R, L), 1)
    idx = aidx * L + bidx

    def cnt_lt(v):
        return jnp.sum(jnp.where(hb < v, 1.0, 0.0))

    def sum_h_lt(v):
        return jnp.sum(jnp.where(hb < v, h, 0.0))

    def ph1(b, v):
        cand = v + jnp.left_shift(jnp.int32(1), 30 - b)
        ok = sum_h_lt(cand) + cnt_lt(cand) - 1.0 <= C
        return jnp.where(ok, cand, v)

    vstar = jax.lax.fori_loop(0, 31, ph1, jnp.int32(0))
    hval = jax.lax.bitcast_convert_type(vstar, jnp.float32)
    n_lt = cnt_lt(vstar)
    s_lt = sum_h_lt(vstar)
    cnt_tie = jnp.sum(jnp.where(hb == vstar, 1.0, 0.0))
    m = jnp.floor((C + 1.0 - n_lt - s_lt) / (hval + 1.0))
    m = jnp.clip(m, 0.0, cnt_tie)
    kstar = n_lt + m
    Sstar = s_lt + m * hval
    total = jnp.sum(h)
    upb = jnp.where(kstar == 0.0, total <= C, Sstar <= C - kstar)
    kf = jnp.minimum(kstar + jnp.where(upb, 1.0, 0.0), jnp.float32(N))

    def ph2(b, v):
        cand = v + jnp.left_shift(jnp.int32(1), 30 - b)
        ok = cnt_lt(cand) < kf
        return jnp.where(ok, cand, v)

    w = jax.lax.fori_loop(0, 31, ph2, jnp.int32(0))
    m2 = kf - cnt_lt(w)
    sum_s_lt = jnp.sum(jnp.where(hb < w, s, 0.0))
    tie = hb == w

    def ph3(_, lo_hi):
        lo, hi = lo_hi
        mid = lo + (hi - lo + 1) // 2
        ok = jnp.sum(jnp.where(tie & (idx <= mid), 1.0, 0.0)) < m2
        return jnp.where(ok, mid, lo), jnp.where(ok, hi, mid - 1)

    jmax, _ = jax.lax.fori_loop(0, 14, ph3, (jnp.int32(-1), jnp.int32(N - 1)))
    sum_s_tie = jnp.sum(jnp.where(tie & (idx <= jmax + 1), s, 0.0))
    res = (sum_s_lt + sum_s_tie) / kf
    out_ref[...] = jnp.full((1, 1), res, dtype=jnp.float32)


def _impl(y_1, t, interpret=False):
    N, K = y_1.shape
    BR = 512
    NB = N // BR
    t2 = t.reshape(N, 1)
    h_all, s_all, w_all = pl.pallas_call(
        functools.partial(_rows_body, K),
        grid=(NB,),
        in_specs=[
            pl.BlockSpec((BR, 1), lambda i: (i, 0)),
            pl.BlockSpec((BR, K), lambda i: (i, 0)),
        ],
        out_specs=[pl.BlockSpec((1, BR, 1), lambda i: (i, 0, 0))] * 3,
        out_shape=[jax.ShapeDtypeStruct((NB, BR, 1), jnp.float32)] * 3,
        interpret=interpret,
    )(t2, y_1)
    R, L = N // 128, 128
    h2 = h_all.reshape(R, L)
    s2 = s_all.reshape(R, L)
    w2 = w_all.reshape(R, L)
    out = pl.pallas_call(
        functools.partial(_sel_body, N, R, L),
        out_shape=jax.ShapeDtypeStruct((1, 1), jnp.float32),
        interpret=interpret,
    )(h2, s2, w2)
    return out[0, 0]


def kernel(y_1, t):
    return _impl(y_1, t)


# speedup vs baseline: 49.0062x; 1.3678x over previous
import functools

import jax
import jax.numpy as jnp
from jax.experimental import pallas as pl
from jax.experimental.pallas import tpu as pltpu


def _stats(y, tcol):
    cols = jax.lax.broadcasted_iota(jnp.int32, y.shape, 1)
    eqm = cols == tcol
    L1 = jnp.sum(jnp.where(eqm, y, 0.0), axis=1, keepdims=True)
    M0 = jnp.max(y, axis=1, keepdims=True)
    M1 = jnp.max(jnp.where(eqm, -jnp.inf, y), axis=1, keepdims=True)
    lse = jnp.log(jnp.sum(jnp.exp(y - M0), axis=1, keepdims=True)) + M0
    f1 = L1 == M0
    h = jnp.maximum(1.0 - L1 + jnp.where(f1, M1, M0), 0.0)
    s = jnp.maximum(1.0 - L1 + jnp.where(f1, M1, lse), 0.0)
    nwrong = jnp.sum(jnp.where(f1, 0.0, 1.0))
    return h, s, nwrong


def _fused_body(N, H, ta_ref, ya_ref, tb_ref, yb_ref, out_ref,
                h_scr, s_scr, e_scr):
    i = pl.program_id(0)

    @pl.when(i == 0)
    def _init():
        e_scr[0] = 0.0

    ha, sa, wa = _stats(ya_ref[...], ta_ref[...])
    hb, sb, wb = _stats(yb_ref[...], tb_ref[...])
    e_scr[0] += wa + wb
    BR = ya_ref.shape[0]
    r16 = BR // 128
    off = h_scr.shape[0] // 2
    h_scr[pl.ds(i * r16, r16), :] = ha.reshape(r16, 128)
    s_scr[pl.ds(i * r16, r16), :] = sa.reshape(r16, 128)
    h_scr[pl.ds(off + i * r16, r16), :] = hb.reshape(r16, 128)
    s_scr[pl.ds(off + i * r16, r16), :] = sb.reshape(r16, 128)

    @pl.when(i == H - 1)
    def _select():
        h = h_scr[...]
        s = s_scr[...]
        R = h.shape[0]
        C = jnp.float32(N) + e_scr[0]
        bits = jax.lax.bitcast_convert_type(h, jnp.int32)
        aidx = jax.lax.broadcasted_iota(jnp.int32, (R, 128), 0)
        bidx = jax.lax.broadcasted_iota(jnp.int32, (R, 128), 1)
        idx = aidx * 128 + bidx

        def cnt_lt(v):
            return jnp.sum(jnp.where(bits < v, 1.0, 0.0))

        def sum_h_lt(v):
            return jnp.sum(jnp.where(bits < v, h, 0.0))

        def ph1(b, v):
            cand = v + jnp.left_shift(jnp.int32(1), 30 - b)
            ok = sum_h_lt(cand) + cnt_lt(cand) - 1.0 <= C
            return jnp.where(ok, cand, v)

        vstar = jax.lax.fori_loop(0, 31, ph1, jnp.int32(0))
        hval = jax.lax.bitcast_convert_type(vstar, jnp.float32)
        n_lt = cnt_lt(vstar)
        s_lt = sum_h_lt(vstar)
        cnt_tie = jnp.sum(jnp.where(bits == vstar, 1.0, 0.0))
        m = jnp.floor((C + 1.0 - n_lt - s_lt) / (hval + 1.0))
        m = jnp.clip(m, 0.0, cnt_tie)
        kstar = n_lt + m
        Sstar = s_lt + m * hval
        total = jnp.sum(h)
        upb = jnp.where(kstar == 0.0, total <= C, Sstar <= C - kstar)
        kf = jnp.minimum(kstar + jnp.where(upb, 1.0, 0.0), jnp.float32(N))
        need = m + (kf - kstar)
        over = need > cnt_tie
        nxt = jnp.min(jnp.where(bits > vstar, bits, jnp.int32(2**31 - 1)))
        w = jnp.where(over, nxt, vstar)
        m2 = jnp.where(over, 1.0, need)
        sum_s_lt = jnp.sum(jnp.where(bits < w, s, 0.0))
        tie = bits == w

        def ph3(_, lo_hi):
            lo, hi = lo_hi
            mid = lo + (hi - lo + 1) // 2
            ok = jnp.sum(jnp.where(tie & (idx <= mid), 1.0, 0.0)) < m2
            return jnp.where(ok, mid, lo), jnp.where(ok, hi, mid - 1)

        jmax, _ = jax.lax.fori_loop(
            0, 15, ph3, (jnp.int32(-1), jnp.int32(N - 1)))
        sum_s_tie = jnp.sum(jnp.where(tie & (idx <= jmax + 1), s, 0.0))
        res = (sum_s_lt + sum_s_tie) / kf
        out_ref[...] = jnp.full((1, 1), res, dtype=jnp.float32)


def _impl(y_1, t, interpret=False):
    N, K = y_1.shape
    BR = 1024
    H = N // BR // 2
    t2 = t.reshape(N, 1)
    out = pl.pallas_call(
        functools.partial(_fused_body, N, H),
        grid=(H,),
        in_specs=[
            pl.BlockSpec((BR, 1), lambda i: (i, 0)),
            pl.BlockSpec((BR, K), lambda i: (i, 0)),
            pl.BlockSpec((BR, 1), lambda i, H=H: (i + H, 0)),
            pl.BlockSpec((BR, K), lambda i, H=H: (i + H, 0)),
        ],
        out_specs=pl.BlockSpec((1, 1), lambda i: (0, 0)),
        out_shape=jax.ShapeDtypeStruct((1, 1), jnp.float32),
        scratch_shapes=[
            pltpu.VMEM((N // 128, 128), jnp.float32),
            pltpu.VMEM((N // 128, 128), jnp.float32),
            pltpu.SMEM((1,), jnp.float32),
        ],
        interpret=interpret,
    )(t2, y_1, t2, y_1)
    return out[0, 0]


def kernel(y_1, t):
    return _impl(y_1, t)
